# Initial kernel scaffold; baseline (speedup 1.0000x reference)
import jax
import jax.numpy as jnp
from jax.experimental import pallas as pl


def kernel(edge_index, X, num_nodes, eps):
    raise NotImplementedError("write your pallas kernel here")


# speedup vs baseline: 3.3900x; 3.3900x over previous
import functools

import jax
import jax.numpy as jnp
from jax import lax
from jax.experimental import pallas as pl
from jax.experimental.pallas import tpu as pltpu
from jax.experimental.pallas import tpu_sc as plsc

NC = 2
NS = 16
CHUNK = 128


def _sc_partial_sums(src2d, dst2d, X, acc_rows, out_rows, chunks_per_tile):
  n_pad, D = X.shape
  mesh = plsc.VectorSubcoreMesh(
      core_axis_name="c", subcore_axis_name="s", num_cores=NC, num_subcores=NS
  )
  rows_init = out_rows // NS

  @functools.partial(
      pl.kernel,
      out_type=jax.ShapeDtypeStruct((NC * out_rows, D), jnp.float32),
      mesh=mesh,
      scratch_types=[
          pltpu.VMEM((chunks_per_tile, CHUNK), jnp.int32),
          pltpu.VMEM((chunks_per_tile, CHUNK), jnp.int32),
          pltpu.VMEM((CHUNK, D), jnp.float32),
          pltpu.VMEM_SHARED((acc_rows, D), jnp.float32),
          pltpu.SemaphoreType.DMA,
      ],
  )
  def k(src_hbm, dst_hbm, x_hbm, out_hbm, src_v, dst_v, rows_v, acc, sem):
    cid = lax.axis_index("c")
    sid = lax.axis_index("s")
    wid = sid * NC + cid

    def zbody(i, c):
      for j in range(D // 16):
        rows_v[i, pl.ds(j * 16, 16)] = jnp.zeros((16,), jnp.float32)
      return c

    lax.fori_loop(0, CHUNK, zbody, 0)
    zslices = acc_rows // (NS * CHUNK)
    for z in range(zslices):
      pltpu.sync_copy(rows_v, acc.at[pl.ds((sid * zslices + z) * CHUNK, CHUNK)])

    base = wid * chunks_per_tile
    pltpu.sync_copy(src_hbm.at[pl.ds(base, chunks_per_tile)], src_v)
    pltpu.sync_copy(dst_hbm.at[pl.ds(base, chunks_per_tile)], dst_v)
    plsc.subcore_barrier()

    def body(j, carry):
      pltpu.async_copy(x_hbm.at[src_v.at[j]], rows_v, sem).wait()
      pltpu.sync_copy(rows_v, acc.at[dst_v.at[j]], add=True)
      return carry

    lax.fori_loop(0, chunks_per_tile, body, 0)
    plsc.subcore_barrier()

    pltpu.sync_copy(
        acc.at[pl.ds(sid * rows_init, rows_init)],
        out_hbm.at[pl.ds(cid * out_rows + sid * rows_init, rows_init)],
    )

  return k(src2d, dst2d, X)


def _combine(X, p0, p1, eps):
  n, D = X.shape
  blk = 1000
  grid = (n + blk - 1) // blk

  def body(eps_ref, x_ref, p0_ref, p1_ref, o_ref):
    o_ref[...] = (1.0 + eps_ref[0]) * x_ref[...] + p0_ref[...] + p1_ref[...]

  return pl.pallas_call(
      body,
      grid=(grid,),
      in_specs=[
          pl.BlockSpec(memory_space=pltpu.SMEM),
          pl.BlockSpec((blk, D), lambda i: (i, 0)),
          pl.BlockSpec((blk, D), lambda i: (i, 0)),
          pl.BlockSpec((blk, D), lambda i: (i, 0)),
      ],
      out_specs=pl.BlockSpec((blk, D), lambda i: (i, 0)),
      out_shape=jax.ShapeDtypeStruct((n, D), jnp.float32),
  )(eps, X, p0, p1)


def kernel(edge_index, X, num_nodes, eps):
  n, D = X.shape
  E = edge_index.shape[1]
  dst = edge_index[0] % n
  src = edge_index[1]

  tile_quant = NC * NS * CHUNK * 8
  e_pad = ((E + tile_quant - 1) // tile_quant) * tile_quant
  chunks_per_tile = e_pad // (NC * NS * CHUNK)
  pad = e_pad - E
  if pad:
    src = jnp.concatenate([src, jnp.zeros((pad,), src.dtype)])
    dst = jnp.concatenate([dst, jnp.full((pad,), n, dst.dtype)])
  src2d = src.reshape(e_pad // CHUNK, CHUNK).astype(jnp.int32)
  dst2d = dst.reshape(e_pad // CHUNK, CHUNK).astype(jnp.int32)

  zquant = NS * CHUNK
  acc_rows = ((n + 1 + zquant - 1) // zquant) * zquant
  oquant = NS * 8
  out_rows = ((n + oquant - 1) // oquant) * oquant

  partial = _sc_partial_sums(src2d, dst2d, X, acc_rows, out_rows, chunks_per_tile)
  p0 = partial[:n]
  p1 = partial[out_rows:out_rows + n]
  return _combine(X, p0, p1, eps.astype(jnp.float32))
